# baseline (speedup 1.0000x reference)
import functools

import jax
import jax.numpy as jnp
import numpy as np
from jax import lax
from jax.experimental import pallas as pl
from jax.experimental.pallas import tpu as pltpu
from jax.experimental.pallas import tpu_sc as plsc

N_NODES = 10000
N_EDGES = 320000
D_IN = 128

NP = 10240
N_CORES = 2
N_SUB = 16
N_TILES = N_CORES * N_SUB
CHUNK = 128
EC = 125
CHUNKS_PER_TILE = 80
ROWS_PER_SUB = NP // N_SUB
ROW_CH = ROWS_PER_SUB // CHUNK


def _make_prop(feat, gather):
    mesh = plsc.VectorSubcoreMesh(core_axis_name="c", subcore_axis_name="s")

    scratch = [
        pltpu.VMEM((CHUNKS_PER_TILE, EC), jnp.int32),
        pltpu.VMEM((CHUNKS_PER_TILE, EC), jnp.int32),
        pltpu.VMEM((CHUNK, feat), jnp.float32),
        pltpu.VMEM((CHUNK, feat), jnp.float32),
        pltpu.VMEM((CHUNK, feat), jnp.float32),
        pltpu.VMEM((CHUNK, feat), jnp.float32),
        pltpu.VMEM((CHUNK, feat), jnp.float32),
        pltpu.VMEM_SHARED((NP, feat), jnp.float32),
        pltpu.VMEM_SHARED((NP, feat), jnp.float32),
        [pltpu.SemaphoreType.DMA] * 4,
        [pltpu.SemaphoreType.DMA] * 4,
    ]

    @functools.partial(
        pl.kernel,
        mesh=mesh,
        out_type=jax.ShapeDtypeStruct((N_CORES, NP, feat), jnp.float32),
        scratch_types=scratch,
        compiler_params=pltpu.CompilerParams(use_tc_tiling_on_sc=False),
    )
    def prop(edges_hbm, h_hbm, const_hbm, out_hbm,
             src_v, dst_v, rows0_v, rows1_v, rows2_v, rows3_v, stage_v,
             acc, h_sh, semg, sems):
        c = lax.axis_index("c")
        s = lax.axis_index("s")
        w = c * N_SUB + s

        pltpu.sync_copy(edges_hbm.at[0].at[w], src_v)
        pltpu.sync_copy(edges_hbm.at[1].at[w], dst_v)
        if gather:
            pltpu.sync_copy(const_hbm, stage_v)
        else:
            pltpu.sync_copy(h_hbm, stage_v)
            pltpu.sync_copy(const_hbm, rows0_v)

        for k in range(ROW_CH):
            r = s * ROWS_PER_SUB + k * CHUNK
            pltpu.sync_copy(stage_v, acc.at[pl.ds(r, CHUNK)])
            if gather:
                pltpu.async_copy(
                    h_hbm.at[pl.ds(r, CHUNK)], rows0_v, semg[0]).wait()
                pltpu.sync_copy(rows0_v, h_sh.at[pl.ds(r, CHUNK)])

        plsc.subcore_barrier()

        bufs = tuple(rv.at[pl.ds(0, EC)]
                     for rv in (rows0_v, rows1_v, rows2_v, rows3_v))

        def g_copy(chunk, b):
            return pltpu.make_async_copy(
                h_sh.at[src_v.at[chunk]], bufs[b], semg[b])

        def s_copy(chunk, b):
            return pltpu.make_async_copy(
                bufs[b], acc.at[dst_v.at[chunk]], sems[b])

        if gather:
            for b in range(4):
                pltpu.async_copy(h_sh.at[src_v.at[b]], bufs[b], semg[b])

            def body(j, carry):
                for b in range(4):
                    cidx = 4 * j + b
                    g_copy(cidx, b).wait()
                    pltpu.async_copy(
                        bufs[b], acc.at[dst_v.at[cidx]], sems[b], add=True)
                    b2 = (b + 2) % 4
                    nxt = cidx + 2

                    @pl.when(jnp.logical_and(nxt >= 4,
                                             nxt < CHUNKS_PER_TILE))
                    def _():
                        s_copy(nxt - 4, b2).wait()
                        pltpu.async_copy(
                            h_sh.at[src_v.at[nxt]], bufs[b2], semg[b2])
                return carry

            lax.fori_loop(0, CHUNKS_PER_TILE // 4, body, 0)

            for ch in range(CHUNKS_PER_TILE - 4, CHUNKS_PER_TILE):
                s_copy(ch, ch % 4).wait()
        else:
            ones_ec = bufs[0]

            def body(j, carry):
                for b in range(2):
                    cidx = 2 * j + b

                    @pl.when(cidx >= 2)
                    def _():
                        pltpu.make_async_copy(
                            ones_ec, acc.at[dst_v.at[cidx - 2]],
                            sems[b]).wait()
                    pltpu.async_copy(
                        ones_ec, acc.at[dst_v.at[cidx]], sems[b], add=True)
                return carry

            lax.fori_loop(0, CHUNKS_PER_TILE // 2, body, 0)
            for ch in (CHUNKS_PER_TILE - 2, CHUNKS_PER_TILE - 1):
                pltpu.make_async_copy(
                    ones_ec, acc.at[dst_v.at[ch]], sems[ch % 2]).wait()

        plsc.subcore_barrier()

        for k in range(ROW_CH):
            r = s * ROWS_PER_SUB + k * CHUNK
            pltpu.sync_copy(acc.at[pl.ds(r, CHUNK)], stage_v)
            pltpu.sync_copy(stage_v, out_hbm.at[c].at[pl.ds(r, CHUNK)])

    return prop


_prop32 = _make_prop(32, gather=True)
_prop16 = _make_prop(16, gather=True)
_deg8 = _make_prop(8, gather=False)


def _stage_a_body(x_ref, pd_hbm, w_ref, o_ref, dis_ref, pd_ref, sem):
    pltpu.make_async_copy(pd_hbm, pd_ref, sem).start()
    pltpu.make_async_copy(pd_hbm, pd_ref, sem).wait()
    deg = 1.0 + pd_ref[0, :, 0:1] + pd_ref[1, :, 0:1]
    d = lax.rsqrt(deg)
    mm = jnp.dot(x_ref[...], w_ref[...], preferred_element_type=jnp.float32)
    o_ref[...] = jnp.concatenate(
        [mm, jnp.zeros((NP - N_NODES, mm.shape[1]), mm.dtype)], axis=0) * d
    dis_ref[...] = jnp.broadcast_to(d, (NP, 8))


def _stage_bc_body(s_hbm, h_ref, dis_ref, b_ref, w_ref, o_ref, s_ref, sem):
    pltpu.make_async_copy(s_hbm, s_ref, sem).start()
    pltpu.make_async_copy(s_hbm, s_ref, sem).wait()
    d = dis_ref[:, 0:1]
    t = jnp.maximum(d * (s_ref[0] + s_ref[1] + h_ref[...]) + b_ref[...], 0.0)
    o_ref[...] = jnp.dot(t, w_ref[...],
                         preferred_element_type=jnp.float32) * d


def _stage_d_body(s_hbm, h_ref, dis_ref, b_ref, o_ref, s_ref, sem):
    pltpu.make_async_copy(s_hbm, s_ref, sem).start()
    pltpu.make_async_copy(s_hbm, s_ref, sem).wait()
    d = dis_ref[:, 0:1]
    t = d * (s_ref[0] + s_ref[1] + h_ref[...]) + b_ref[...]
    o_ref[...] = t[:N_NODES, :8]


def _full_spec(shape):
    return pl.BlockSpec(shape, lambda: tuple(0 for _ in shape))


def _any_spec():
    return pl.BlockSpec(memory_space=pl.ANY)


def _stage_a(x, pdeg, w):
    return pl.pallas_call(
        _stage_a_body,
        in_specs=[_full_spec(x.shape), _any_spec(), _full_spec(w.shape)],
        out_specs=(_full_spec((NP, w.shape[1])), _full_spec((NP, 8))),
        out_shape=(jax.ShapeDtypeStruct((NP, w.shape[1]), jnp.float32),
                   jax.ShapeDtypeStruct((NP, 8), jnp.float32)),
        scratch_shapes=[pltpu.VMEM(pdeg.shape, jnp.float32),
                        pltpu.SemaphoreType.DMA],
    )(x, pdeg, w)


def _stage_bc(sp, h, dis, b, w):
    f_out = w.shape[1]
    return pl.pallas_call(
        _stage_bc_body,
        in_specs=[_any_spec(), _full_spec(h.shape),
                  _full_spec(dis.shape), _full_spec(b.shape),
                  _full_spec(w.shape)],
        out_specs=_full_spec((NP, f_out)),
        out_shape=jax.ShapeDtypeStruct((NP, f_out), jnp.float32),
        scratch_shapes=[pltpu.VMEM(sp.shape, jnp.float32),
                        pltpu.SemaphoreType.DMA],
    )(sp, h, dis, b, w)


def _stage_d(sp, h, dis, b):
    return pl.pallas_call(
        _stage_d_body,
        in_specs=[_any_spec(), _full_spec(h.shape),
                  _full_spec(dis.shape), _full_spec(b.shape)],
        out_specs=_full_spec((N_NODES, 8)),
        out_shape=jax.ShapeDtypeStruct((N_NODES, 8), jnp.float32),
        scratch_shapes=[pltpu.VMEM(sp.shape, jnp.float32),
                        pltpu.SemaphoreType.DMA],
    )(sp, h, dis, b)


def kernel(x, edge_index, W1, b1, W2, b2, W3, b3):
    edges = edge_index.astype(jnp.int32).reshape(
        2, N_TILES, CHUNKS_PER_TILE, EC)

    W1p = jnp.pad(W1, ((0, 0), (0, 32 - W1.shape[1])))
    b1p = jnp.pad(b1, (0, 32 - b1.shape[0])).reshape(1, 32)
    W2p = jnp.pad(W2, ((0, 32 - W2.shape[0]), (0, 16 - W2.shape[1])))
    b2p = jnp.pad(b2, (0, 16 - b2.shape[0])).reshape(1, 16)
    W3p = jnp.pad(W3, ((0, 16 - W3.shape[0]), (0, 16 - W3.shape[1])))
    b3p = jnp.pad(b3, (0, 16 - b3.shape[0])).reshape(1, 16)

    zeros32 = jnp.asarray(np.zeros((CHUNK, 32), np.float32))
    zeros16 = jnp.asarray(np.zeros((CHUNK, 16), np.float32))
    zeros8 = jnp.asarray(np.zeros((CHUNK, 8), np.float32))
    ones8 = jnp.asarray(np.ones((CHUNK, 8), np.float32))

    pdeg = _deg8(edges, zeros8, ones8)

    h1, dis = _stage_a(x, pdeg, W1p)
    s1 = _prop32(edges, h1, zeros32)
    h2 = _stage_bc(s1, h1, dis, b1p, W2p)
    s2 = _prop16(edges, h2, zeros16)
    h3 = _stage_bc(s2, h2, dis, b2p, W3p)
    s3 = _prop16(edges, h3, zeros16)
    return _stage_d(s3, h3, dis, b3p)


# speedup vs baseline: 1.0579x; 1.0579x over previous
import functools

import jax
import jax.numpy as jnp
import numpy as np
from jax import lax
from jax.experimental import pallas as pl
from jax.experimental.pallas import tpu as pltpu
from jax.experimental.pallas import tpu_sc as plsc

N_NODES = 10000
N_EDGES = 320000
D_IN = 128

NP = 10240
N_CORES = 2
N_SUB = 16
N_TILES = N_CORES * N_SUB
CHUNK = 128
EC = 125
CHUNKS_PER_TILE = 80
ROWS_PER_SUB = NP // N_SUB
ROW_CH = ROWS_PER_SUB // CHUNK


def _make_prop(feat, gather):
    mesh = plsc.VectorSubcoreMesh(core_axis_name="c", subcore_axis_name="s")

    scratch = [
        pltpu.VMEM((CHUNKS_PER_TILE, EC), jnp.int32),
        pltpu.VMEM((CHUNKS_PER_TILE, EC), jnp.int32),
        pltpu.VMEM((CHUNK, feat), jnp.float32),
        pltpu.VMEM((CHUNK, feat), jnp.float32),
        pltpu.VMEM((CHUNK, feat), jnp.float32),
        pltpu.VMEM((CHUNK, feat), jnp.float32),
        pltpu.VMEM((CHUNK, feat), jnp.float32),
        pltpu.VMEM_SHARED((NP, feat), jnp.float32),
        pltpu.VMEM_SHARED((NP, feat), jnp.float32),
        [pltpu.SemaphoreType.DMA] * 4,
        [pltpu.SemaphoreType.DMA] * 4,
    ]

    @functools.partial(
        pl.kernel,
        mesh=mesh,
        out_type=jax.ShapeDtypeStruct((N_CORES, NP, feat), jnp.float32),
        scratch_types=scratch,
        compiler_params=pltpu.CompilerParams(use_tc_tiling_on_sc=False),
    )
    def prop(edges_hbm, h_hbm, const_hbm, out_hbm,
             src_v, dst_v, rows0_v, rows1_v, rows2_v, rows3_v, stage_v,
             acc, h_sh, semg, sems):
        c = lax.axis_index("c")
        s = lax.axis_index("s")
        w = c * N_SUB + s

        pltpu.async_copy(edges_hbm.at[0].at[w], src_v, sems[0])
        pltpu.async_copy(edges_hbm.at[1].at[w], dst_v, sems[1])
        if gather:
            pltpu.sync_copy(const_hbm, stage_v)
        else:
            pltpu.sync_copy(h_hbm, stage_v)
            pltpu.sync_copy(const_hbm, rows0_v)

        for k in range(ROW_CH):
            r = s * ROWS_PER_SUB + k * CHUNK
            pltpu.async_copy(stage_v, acc.at[pl.ds(r, CHUNK)], sems[2])
        if gather:
            for k in range(ROW_CH):
                r = s * ROWS_PER_SUB + k * CHUNK
                pltpu.async_copy(
                    h_hbm.at[pl.ds(r, CHUNK)], rows1_v, semg[3]).wait()
                pltpu.sync_copy(rows1_v, h_sh.at[pl.ds(r, CHUNK)])
        pltpu.make_async_copy(edges_hbm.at[0].at[w], src_v, sems[0]).wait()
        pltpu.make_async_copy(edges_hbm.at[1].at[w], dst_v, sems[1]).wait()
        for k in range(ROW_CH):
            r = s * ROWS_PER_SUB + k * CHUNK
            pltpu.make_async_copy(
                stage_v, acc.at[pl.ds(r, CHUNK)], sems[2]).wait()

        plsc.subcore_barrier()

        bufs = tuple(rv.at[pl.ds(0, EC)]
                     for rv in (rows0_v, rows1_v, rows2_v, rows3_v))

        def g_copy(chunk, b):
            return pltpu.make_async_copy(
                h_sh.at[src_v.at[chunk]], bufs[b], semg[b])

        def s_copy(chunk, b):
            return pltpu.make_async_copy(
                bufs[b], acc.at[dst_v.at[chunk]], sems[b])

        if gather:
            for b in range(4):
                pltpu.async_copy(h_sh.at[src_v.at[b]], bufs[b], semg[b])

            def body(j, carry):
                for b in range(4):
                    cidx = 4 * j + b
                    g_copy(cidx, b).wait()
                    pltpu.async_copy(
                        bufs[b], acc.at[dst_v.at[cidx]], sems[b], add=True)
                    b2 = (b + 2) % 4
                    nxt = cidx + 2

                    @pl.when(jnp.logical_and(nxt >= 4,
                                             nxt < CHUNKS_PER_TILE))
                    def _():
                        s_copy(nxt - 4, b2).wait()
                        pltpu.async_copy(
                            h_sh.at[src_v.at[nxt]], bufs[b2], semg[b2])
                return carry

            lax.fori_loop(0, CHUNKS_PER_TILE // 4, body, 0)

            for ch in range(CHUNKS_PER_TILE - 4, CHUNKS_PER_TILE):
                s_copy(ch, ch % 4).wait()
        else:
            ones_ec = bufs[0]

            def body(j, carry):
                for b in range(2):
                    cidx = 2 * j + b

                    @pl.when(cidx >= 2)
                    def _():
                        pltpu.make_async_copy(
                            ones_ec, acc.at[dst_v.at[cidx - 2]],
                            sems[b]).wait()
                    pltpu.async_copy(
                        ones_ec, acc.at[dst_v.at[cidx]], sems[b], add=True)
                return carry

            lax.fori_loop(0, CHUNKS_PER_TILE // 2, body, 0)
            for ch in (CHUNKS_PER_TILE - 2, CHUNKS_PER_TILE - 1):
                pltpu.make_async_copy(
                    ones_ec, acc.at[dst_v.at[ch]], sems[ch % 2]).wait()

        plsc.subcore_barrier()

        wbufs = (rows0_v, rows1_v, rows2_v, rows3_v, stage_v)
        wsems = (sems[0], sems[1], sems[2], sems[3], semg[0])
        for k in range(ROW_CH):
            r = s * ROWS_PER_SUB + k * CHUNK
            pltpu.async_copy(acc.at[pl.ds(r, CHUNK)], wbufs[k], wsems[k])
        for k in range(ROW_CH):
            r = s * ROWS_PER_SUB + k * CHUNK
            pltpu.make_async_copy(
                acc.at[pl.ds(r, CHUNK)], wbufs[k], wsems[k]).wait()
            pltpu.async_copy(
                wbufs[k], out_hbm.at[c].at[pl.ds(r, CHUNK)], wsems[k])
        for k in range(ROW_CH):
            r = s * ROWS_PER_SUB + k * CHUNK
            pltpu.make_async_copy(
                wbufs[k], out_hbm.at[c].at[pl.ds(r, CHUNK)], wsems[k]).wait()

    return prop


_prop32 = _make_prop(32, gather=True)
_prop16 = _make_prop(16, gather=True)
_deg8 = _make_prop(8, gather=False)


def _stage_a0_body(x_ref, w_ref, o_ref):
    mm = jnp.dot(x_ref[...], w_ref[...], preferred_element_type=jnp.float32)
    o_ref[...] = jnp.concatenate(
        [mm, jnp.zeros((NP - N_NODES, mm.shape[1]), mm.dtype)], axis=0)


def _stage_a1_body(mm_ref, pd_ref, o_ref, dis_ref):
    deg = 1.0 + pd_ref[0, :, 0:1] + pd_ref[1, :, 0:1]
    d = lax.rsqrt(deg)
    o_ref[...] = mm_ref[...] * d
    dis_ref[...] = jnp.broadcast_to(d, (NP, 8))


def _stage_bc_body(s_ref, h_ref, dis_ref, b_ref, w_ref, o_ref):
    d = dis_ref[:, 0:1]
    t = jnp.maximum(d * (s_ref[0] + s_ref[1] + h_ref[...]) + b_ref[...], 0.0)
    o_ref[...] = jnp.dot(t, w_ref[...],
                         preferred_element_type=jnp.float32) * d


def _stage_d_body(s_ref, h_ref, dis_ref, b_ref, o_ref):
    d = dis_ref[:, 0:1]
    t = d * (s_ref[0] + s_ref[1] + h_ref[...]) + b_ref[...]
    o_ref[...] = t[:N_NODES, :8]


def _full_spec(shape):
    return pl.BlockSpec(shape, lambda: tuple(0 for _ in shape))


def _stage_a0(x, w):
    return pl.pallas_call(
        _stage_a0_body,
        in_specs=[_full_spec(x.shape), _full_spec(w.shape)],
        out_specs=_full_spec((NP, w.shape[1])),
        out_shape=jax.ShapeDtypeStruct((NP, w.shape[1]), jnp.float32),
    )(x, w)


def _stage_a1(mm, pdeg):
    return pl.pallas_call(
        _stage_a1_body,
        in_specs=[_full_spec(mm.shape), _full_spec(pdeg.shape)],
        out_specs=(_full_spec(mm.shape), _full_spec((NP, 8))),
        out_shape=(jax.ShapeDtypeStruct(mm.shape, jnp.float32),
                   jax.ShapeDtypeStruct((NP, 8), jnp.float32)),
    )(mm, pdeg)


def _stage_bc(sp, h, dis, b, w):
    f_out = w.shape[1]
    return pl.pallas_call(
        _stage_bc_body,
        in_specs=[_full_spec(sp.shape), _full_spec(h.shape),
                  _full_spec(dis.shape), _full_spec(b.shape),
                  _full_spec(w.shape)],
        out_specs=_full_spec((NP, f_out)),
        out_shape=jax.ShapeDtypeStruct((NP, f_out), jnp.float32),
    )(sp, h, dis, b, w)


def _stage_d(sp, h, dis, b):
    return pl.pallas_call(
        _stage_d_body,
        in_specs=[_full_spec(sp.shape), _full_spec(h.shape),
                  _full_spec(dis.shape), _full_spec(b.shape)],
        out_specs=_full_spec((N_NODES, 8)),
        out_shape=jax.ShapeDtypeStruct((N_NODES, 8), jnp.float32),
    )(sp, h, dis, b)


def kernel(x, edge_index, W1, b1, W2, b2, W3, b3):
    edges = edge_index.astype(jnp.int32).reshape(
        2, N_TILES, CHUNKS_PER_TILE, EC)

    W1p = jnp.pad(W1, ((0, 0), (0, 32 - W1.shape[1])))
    b1p = jnp.pad(b1, (0, 32 - b1.shape[0])).reshape(1, 32)
    W2p = jnp.pad(W2, ((0, 32 - W2.shape[0]), (0, 16 - W2.shape[1])))
    b2p = jnp.pad(b2, (0, 16 - b2.shape[0])).reshape(1, 16)
    W3p = jnp.pad(W3, ((0, 16 - W3.shape[0]), (0, 16 - W3.shape[1])))
    b3p = jnp.pad(b3, (0, 16 - b3.shape[0])).reshape(1, 16)

    zeros32 = jnp.asarray(np.zeros((CHUNK, 32), np.float32))
    zeros16 = jnp.asarray(np.zeros((CHUNK, 16), np.float32))
    zeros8 = jnp.asarray(np.zeros((CHUNK, 8), np.float32))
    ones8 = jnp.asarray(np.ones((CHUNK, 8), np.float32))

    mm = _stage_a0(x, W1p)
    pdeg = _deg8(edges, zeros8, ones8)

    h1, dis = _stage_a1(mm, pdeg)
    s1 = _prop32(edges, h1, zeros32)
    h2 = _stage_bc(s1, h1, dis, b1p, W2p)
    s2 = _prop16(edges, h2, zeros16)
    h3 = _stage_bc(s2, h2, dis, b2p, W3p)
    s3 = _prop16(edges, h3, zeros16)
    return _stage_d(s3, h3, dis, b3p)


# speedup vs baseline: 1.0601x; 1.0021x over previous
import functools

import jax
import jax.numpy as jnp
import numpy as np
from jax import lax
from jax.experimental import pallas as pl
from jax.experimental.pallas import tpu as pltpu
from jax.experimental.pallas import tpu_sc as plsc

N_NODES = 10000
N_EDGES = 320000
D_IN = 128

NP = 10240
N_CORES = 2
N_SUB = 16
N_TILES = N_CORES * N_SUB
CHUNK = 128
EC = 125
CHUNKS_PER_TILE = 80
ROWS_PER_SUB = NP // N_SUB
ROW_CH = ROWS_PER_SUB // CHUNK


def _make_prop(feat, gather):
    mesh = plsc.VectorSubcoreMesh(core_axis_name="c", subcore_axis_name="s")

    scratch = [
        pltpu.VMEM((CHUNKS_PER_TILE, EC), jnp.int32),
        pltpu.VMEM((CHUNKS_PER_TILE, EC), jnp.int32),
        pltpu.VMEM((CHUNK, feat), jnp.float32),
        pltpu.VMEM((CHUNK, feat), jnp.float32),
        pltpu.VMEM((CHUNK, feat), jnp.float32),
        pltpu.VMEM((CHUNK, feat), jnp.float32),
        pltpu.VMEM((CHUNK, feat), jnp.float32),
        pltpu.VMEM((CHUNK, 128), jnp.float32),
        pltpu.VMEM_SHARED((NP, feat), jnp.float32),
        pltpu.VMEM_SHARED((NP, feat), jnp.float32),
        [pltpu.SemaphoreType.DMA] * 4,
        [pltpu.SemaphoreType.DMA] * 4,
    ]

    @functools.partial(
        pl.kernel,
        mesh=mesh,
        out_type=jax.ShapeDtypeStruct((N_CORES, NP, feat), jnp.float32),
        scratch_types=scratch,
        compiler_params=pltpu.CompilerParams(use_tc_tiling_on_sc=False),
    )
    def prop(edges_hbm, h_hbm, const_hbm, out_hbm,
             src_v, dst_v, rows0_v, rows1_v, rows2_v, rows3_v, stage_v,
             wide_v, acc, h_sh, semg, sems):
        c = lax.axis_index("c")
        s = lax.axis_index("s")
        w = c * N_SUB + s

        pltpu.async_copy(edges_hbm.at[0].at[w], src_v, sems[0])
        pltpu.async_copy(edges_hbm.at[1].at[w], dst_v, sems[1])
        if gather:
            pltpu.sync_copy(const_hbm, stage_v)
        else:
            pltpu.sync_copy(h_hbm, stage_v)
            pltpu.sync_copy(const_hbm, rows0_v)

        for k in range(ROW_CH):
            r = s * ROWS_PER_SUB + k * CHUNK
            pltpu.async_copy(stage_v, acc.at[pl.ds(r, CHUNK)], sems[2])
        if gather:
            for k in range(ROW_CH):
                r = s * ROWS_PER_SUB + k * CHUNK
                pltpu.async_copy(
                    h_hbm.at[pl.ds(r, CHUNK)], wide_v, semg[3]).wait()
                pltpu.sync_copy(wide_v.at[:, pl.ds(0, feat)],
                                h_sh.at[pl.ds(r, CHUNK)])
        pltpu.make_async_copy(edges_hbm.at[0].at[w], src_v, sems[0]).wait()
        pltpu.make_async_copy(edges_hbm.at[1].at[w], dst_v, sems[1]).wait()
        for k in range(ROW_CH):
            r = s * ROWS_PER_SUB + k * CHUNK
            pltpu.make_async_copy(
                stage_v, acc.at[pl.ds(r, CHUNK)], sems[2]).wait()

        plsc.subcore_barrier()

        bufs = tuple(rv.at[pl.ds(0, EC)]
                     for rv in (rows0_v, rows1_v, rows2_v, rows3_v))

        def g_copy(chunk, b):
            return pltpu.make_async_copy(
                h_sh.at[src_v.at[chunk]], bufs[b], semg[b])

        def s_copy(chunk, b):
            return pltpu.make_async_copy(
                bufs[b], acc.at[dst_v.at[chunk]], sems[b])

        if gather:
            for b in range(4):
                pltpu.async_copy(h_sh.at[src_v.at[b]], bufs[b], semg[b])

            def body(j, carry):
                for b in range(4):
                    cidx = 4 * j + b
                    g_copy(cidx, b).wait()
                    pltpu.async_copy(
                        bufs[b], acc.at[dst_v.at[cidx]], sems[b], add=True)
                    b2 = (b + 2) % 4
                    nxt = cidx + 2

                    @pl.when(jnp.logical_and(nxt >= 4,
                                             nxt < CHUNKS_PER_TILE))
                    def _():
                        s_copy(nxt - 4, b2).wait()
                        pltpu.async_copy(
                            h_sh.at[src_v.at[nxt]], bufs[b2], semg[b2])
                return carry

            lax.fori_loop(0, CHUNKS_PER_TILE // 4, body, 0)

            for ch in range(CHUNKS_PER_TILE - 4, CHUNKS_PER_TILE):
                s_copy(ch, ch % 4).wait()
        else:
            ones_ec = bufs[0]

            def body(j, carry):
                for b in range(2):
                    cidx = 2 * j + b

                    @pl.when(cidx >= 2)
                    def _():
                        pltpu.make_async_copy(
                            ones_ec, acc.at[dst_v.at[cidx - 2]],
                            sems[b]).wait()
                    pltpu.async_copy(
                        ones_ec, acc.at[dst_v.at[cidx]], sems[b], add=True)
                return carry

            lax.fori_loop(0, CHUNKS_PER_TILE // 2, body, 0)
            for ch in (CHUNKS_PER_TILE - 2, CHUNKS_PER_TILE - 1):
                pltpu.make_async_copy(
                    ones_ec, acc.at[dst_v.at[ch]], sems[ch % 2]).wait()

        plsc.subcore_barrier()

        wbufs = (rows0_v, rows1_v, rows2_v, rows3_v, stage_v)
        wsems = (sems[0], sems[1], sems[2], sems[3], semg[0])
        for k in range(ROW_CH):
            r = s * ROWS_PER_SUB + k * CHUNK
            pltpu.async_copy(acc.at[pl.ds(r, CHUNK)], wbufs[k], wsems[k])
        for k in range(ROW_CH):
            r = s * ROWS_PER_SUB + k * CHUNK
            pltpu.make_async_copy(
                acc.at[pl.ds(r, CHUNK)], wbufs[k], wsems[k]).wait()
            pltpu.async_copy(
                wbufs[k], out_hbm.at[c].at[pl.ds(r, CHUNK)], wsems[k])
        for k in range(ROW_CH):
            r = s * ROWS_PER_SUB + k * CHUNK
            pltpu.make_async_copy(
                wbufs[k], out_hbm.at[c].at[pl.ds(r, CHUNK)], wsems[k]).wait()

    return prop


_prop32 = _make_prop(32, gather=True)
_prop16 = _make_prop(16, gather=True)
_deg8 = _make_prop(8, gather=False)


def _stage_a0_body(x_ref, w_ref, o_ref):
    mm = jnp.dot(x_ref[...], w_ref[...], preferred_element_type=jnp.float32)
    o_ref[...] = jnp.concatenate(
        [mm, jnp.zeros((NP - N_NODES, mm.shape[1]), mm.dtype)], axis=0)


def _pad128(t):
    return jnp.concatenate(
        [t, jnp.zeros((t.shape[0], 128 - t.shape[1]), t.dtype)], axis=1)


def _stage_a1_body(mm_ref, pd_ref, o_ref, dis_ref):
    deg = 1.0 + pd_ref[0, :, 0:1] + pd_ref[1, :, 0:1]
    d = lax.rsqrt(deg)
    o_ref[...] = _pad128(mm_ref[...] * d)
    dis_ref[...] = jnp.broadcast_to(d, (NP, 8))


def _stage_bc_body(s_ref, h_ref, dis_ref, b_ref, w_ref, o_ref):
    d = dis_ref[:, 0:1]
    f_in = w_ref.shape[0]
    t = jnp.maximum(
        d * (s_ref[0] + s_ref[1] + h_ref[:, :f_in]) + b_ref[...], 0.0)
    o_ref[...] = _pad128(
        jnp.dot(t, w_ref[...], preferred_element_type=jnp.float32) * d)


def _stage_d_body(s_ref, h_ref, dis_ref, b_ref, o_ref):
    d = dis_ref[:, 0:1]
    t = d * (s_ref[0] + s_ref[1] + h_ref[:, :16]) + b_ref[...]
    o_ref[...] = t[:N_NODES, :8]


def _full_spec(shape):
    return pl.BlockSpec(shape, lambda: tuple(0 for _ in shape))


def _stage_a0(x, w):
    return pl.pallas_call(
        _stage_a0_body,
        in_specs=[_full_spec(x.shape), _full_spec(w.shape)],
        out_specs=_full_spec((NP, w.shape[1])),
        out_shape=jax.ShapeDtypeStruct((NP, w.shape[1]), jnp.float32),
    )(x, w)


def _stage_a1(mm, pdeg):
    return pl.pallas_call(
        _stage_a1_body,
        in_specs=[_full_spec(mm.shape), _full_spec(pdeg.shape)],
        out_specs=(_full_spec((NP, 128)), _full_spec((NP, 8))),
        out_shape=(jax.ShapeDtypeStruct((NP, 128), jnp.float32),
                   jax.ShapeDtypeStruct((NP, 8), jnp.float32)),
    )(mm, pdeg)


def _stage_bc(sp, h, dis, b, w):
    return pl.pallas_call(
        _stage_bc_body,
        in_specs=[_full_spec(sp.shape), _full_spec(h.shape),
                  _full_spec(dis.shape), _full_spec(b.shape),
                  _full_spec(w.shape)],
        out_specs=_full_spec((NP, 128)),
        out_shape=jax.ShapeDtypeStruct((NP, 128), jnp.float32),
    )(sp, h, dis, b, w)


def _stage_d(sp, h, dis, b):
    return pl.pallas_call(
        _stage_d_body,
        in_specs=[_full_spec(sp.shape), _full_spec(h.shape),
                  _full_spec(dis.shape), _full_spec(b.shape)],
        out_specs=_full_spec((N_NODES, 8)),
        out_shape=jax.ShapeDtypeStruct((N_NODES, 8), jnp.float32),
    )(sp, h, dis, b)


def kernel(x, edge_index, W1, b1, W2, b2, W3, b3):
    edges = edge_index.astype(jnp.int32).reshape(
        2, N_TILES, CHUNKS_PER_TILE, EC)

    W1p = jnp.pad(W1, ((0, 0), (0, 32 - W1.shape[1])))
    b1p = jnp.pad(b1, (0, 32 - b1.shape[0])).reshape(1, 32)
    W2p = jnp.pad(W2, ((0, 32 - W2.shape[0]), (0, 16 - W2.shape[1])))
    b2p = jnp.pad(b2, (0, 16 - b2.shape[0])).reshape(1, 16)
    W3p = jnp.pad(W3, ((0, 16 - W3.shape[0]), (0, 16 - W3.shape[1])))
    b3p = jnp.pad(b3, (0, 16 - b3.shape[0])).reshape(1, 16)

    zeros32 = jnp.asarray(np.zeros((CHUNK, 32), np.float32))
    zeros16 = jnp.asarray(np.zeros((CHUNK, 16), np.float32))
    zeros8 = jnp.asarray(np.zeros((CHUNK, 8), np.float32))
    ones8 = jnp.asarray(np.ones((CHUNK, 8), np.float32))

    mm = _stage_a0(x, W1p)
    pdeg = _deg8(edges, zeros8, ones8)

    h1, dis = _stage_a1(mm, pdeg)
    s1 = _prop32(edges, h1, zeros32)
    h2 = _stage_bc(s1, h1, dis, b1p, W2p)
    s2 = _prop16(edges, h2, zeros16)
    h3 = _stage_bc(s2, h2, dis, b2p, W3p)
    s3 = _prop16(edges, h3, zeros16)
    return _stage_d(s3, h3, dis, b3p)
